# Initial kernel scaffold; baseline (speedup 1.0000x reference)
import jax
import jax.numpy as jnp
from jax.experimental import pallas as pl


def kernel(x, pos, batch, sa1_W1, sa1_b1, sa1_W2, sa1_b2, sa2_W1, sa2_b1, sa2_W2, sa2_b2, sa3_W1, sa3_b1, sa3_W2, sa3_b2, fp3_W1, fp3_b1, fp3_W2, fp3_b2, fp2_W1, fp2_b1, fp2_W2, fp2_b2, fp1_W1, fp1_b1, fp1_W2, fp1_b2, fp1_W3, fp1_b3, mlp_W1, mlp_b1, mlp_W2, mlp_b2, mlp_W3, mlp_b3):
    raise NotImplementedError("write your pallas kernel here")


# speedup vs baseline: 2.0992x; 2.0992x over previous
import functools

import jax
import jax.numpy as jnp
from jax.experimental import pallas as pl
from jax.experimental.pallas import tpu as pltpu

N = 50000
R1, R2 = 0.2, 0.4
KNEI = 64
M1, M2 = 10000, 2500
OUT_DIM = 3
CHUNK = 500


def _fps_body(n_samples, n_valid, px_ref, py_ref, pz_ref, out_ref, dist_ref):
    Rr, Ll = px_ref.shape
    rows = jax.lax.broadcasted_iota(jnp.int32, (Rr, Ll), 0)
    cols = jax.lax.broadcasted_iota(jnp.int32, (Rr, Ll), 1)
    flat = rows * Ll + cols
    valid = flat < n_valid

    px0 = px_ref[0, 0]
    py0 = py_ref[0, 0]
    pz0 = pz_ref[0, 0]
    d0 = (px_ref[...] - px0) ** 2 + (py_ref[...] - py0) ** 2 + (pz_ref[...] - pz0) ** 2
    dist_ref[...] = jnp.where(valid, d0, -1.0)
    out_ref[pl.ds(0, 1), :] = jnp.zeros((1, 1), jnp.int32)

    def body(i, carry):
        dist = dist_ref[...]
        m = jnp.max(dist)
        nxt = jnp.min(jnp.where(dist == m, flat, jnp.int32(2**31 - 1)))
        out_ref[pl.ds(i, 1), :] = jnp.full((1, 1), nxt, jnp.int32)
        sel = flat == nxt
        px = jnp.sum(jnp.where(sel, px_ref[...], 0.0))
        py = jnp.sum(jnp.where(sel, py_ref[...], 0.0))
        pz = jnp.sum(jnp.where(sel, pz_ref[...], 0.0))
        nd = (px_ref[...] - px) ** 2 + (py_ref[...] - py) ** 2 + (pz_ref[...] - pz) ** 2
        dist_ref[...] = jnp.minimum(dist, nd)
        return carry

    jax.lax.fori_loop(1, n_samples, body, 0)


def _fps_pallas(pos, n_samples):
    n = pos.shape[0]
    Ll = 128
    Rr = -(-n // Ll)
    Rr = -(-Rr // 8) * 8
    pad = Rr * Ll - n
    px = jnp.pad(pos[:, 0], (0, pad), constant_values=1e9).reshape(Rr, Ll)
    py = jnp.pad(pos[:, 1], (0, pad), constant_values=1e9).reshape(Rr, Ll)
    pz = jnp.pad(pos[:, 2], (0, pad), constant_values=1e9).reshape(Rr, Ll)
    out = pl.pallas_call(
        functools.partial(_fps_body, n_samples, n),
        out_shape=jax.ShapeDtypeStruct((n_samples, 1), jnp.int32),
        scratch_shapes=[pltpu.VMEM((Rr, Ll), jnp.float32)],
    )(px, py, pz)
    return out[:, 0]


def _mlp_j(h, params):
    n = len(params)
    for i, (W, b) in enumerate(params):
        h = h @ W + b
        if i < n - 1:
            h = jax.nn.relu(h)
    return h


def _knn_j(q, src, k, chunk):
    M = q.shape[0]
    qc = q.reshape(M // chunk, chunk, q.shape[1])

    def f(qb):
        d2 = jnp.sum((qb[:, None, :] - src[None, :, :]) ** 2, axis=-1)
        negv, idx = jax.lax.top_k(-d2, k)
        return (-negv, idx)

    d2s, idxs = jax.lax.map(f, qc)
    return d2s.reshape(M, k), idxs.reshape(M, k)


def _sa_conv_j(x, pos, q_idx, nbr_idx, mask, params):
    q_pos = pos[q_idx]
    x_j = x[nbr_idx]
    rel = pos[nbr_idx] - q_pos[:, None, :]
    h = _mlp_j(jnp.concatenate([x_j, rel], axis=-1), params)
    h = jnp.where(mask[..., None], h, -1e30)
    return jnp.max(h, axis=1)


def _interp_j(x_src, pos_src, pos_q, nbr_idx):
    d2 = jnp.sum((pos_q[:, None, :] - pos_src[nbr_idx]) ** 2, axis=-1)
    w = 1.0 / jnp.maximum(d2, 1e-16)
    num = jnp.sum(w[..., None] * x_src[nbr_idx], axis=1)
    return num / jnp.sum(w, axis=1, keepdims=True)


def kernel(x, pos, batch,
           sa1_W1, sa1_b1, sa1_W2, sa1_b2,
           sa2_W1, sa2_b1, sa2_W2, sa2_b2,
           sa3_W1, sa3_b1, sa3_W2, sa3_b2,
           fp3_W1, fp3_b1, fp3_W2, fp3_b2,
           fp2_W1, fp2_b1, fp2_W2, fp2_b2,
           fp1_W1, fp1_b1, fp1_W2, fp1_b2, fp1_W3, fp1_b3,
           mlp_W1, mlp_b1, mlp_W2, mlp_b2, mlp_W3, mlp_b3):
    wd = {k: v for k, v in locals().items() if k not in ('x', 'pos', 'batch')}
    P = lambda n, L: [(wd[n + '_W' + str(j)], wd[n + '_b' + str(j)]) for j in range(1, L + 1)]

    idx1 = _fps_pallas(pos, M1)
    pos1 = pos[idx1]
    d2a, nbr1 = _knn_j(pos1, pos, KNEI, CHUNK)
    mask1 = d2a <= R1 * R1
    idx2 = _fps_pallas(pos1, M2)
    pos2 = pos1[idx2]
    d2b, nbr2 = _knn_j(pos2, pos1, KNEI, CHUNK)
    mask2 = d2b <= R2 * R2
    k2 = _knn_j(pos1, pos2, 3, CHUNK)[1]
    k1 = _knn_j(pos, pos1, 3, CHUNK)[1]

    x1 = _sa_conv_j(x, pos, idx1, nbr1, mask1, P('sa1', 2))
    x2 = _sa_conv_j(x1, pos1, idx2, nbr2, mask2, P('sa2', 2))
    h = _mlp_j(jnp.concatenate([x2, pos2], axis=-1), P('sa3', 2))
    xg = jnp.max(h, axis=0, keepdims=True)
    posg = jnp.zeros((1, 3), dtype=pos.dtype)
    xi = _interp_j(xg, posg, pos2, jnp.zeros((M2, 1), dtype=jnp.int32))
    xf = _mlp_j(jnp.concatenate([xi, x2], axis=-1), P('fp3', 2))
    xi = _interp_j(xf, pos2, pos1, k2)
    xf = _mlp_j(jnp.concatenate([xi, x1], axis=-1), P('fp2', 2))
    xi = _interp_j(xf, pos1, pos, k1)
    xf = _mlp_j(jnp.concatenate([xi, x], axis=-1), P('fp1', 3))
    return _mlp_j(xf, P('mlp', 3))
